# Initial kernel scaffold; baseline (speedup 1.0000x reference)
import jax
import jax.numpy as jnp
from jax.experimental import pallas as pl


def kernel(x, edge_index, W, b):
    raise NotImplementedError("write your pallas kernel here")


# speedup vs baseline: 3.1100x; 3.1100x over previous
import functools

import jax
import jax.numpy as jnp
from jax import lax
from jax.experimental import pallas as pl
from jax.experimental.pallas import tpu as pltpu
from jax.experimental.pallas import tpu_sc as plsc

_NC = 2
_NS = 16
_NW = _NC * _NS
_CHUNK = 128
_GROUP = 16


def _sc_aggregate(n_pad, cpw, x, srcp, dstp, zeros):
    d = x.shape[1]
    rows_pt = n_pad // _NS
    mesh = plsc.VectorSubcoreMesh(core_axis_name="c", subcore_axis_name="s")

    @functools.partial(
        pl.kernel,
        out_type=(
            jax.ShapeDtypeStruct((n_pad, d), jnp.float32),
            jax.ShapeDtypeStruct((n_pad, d), jnp.float32),
        ),
        mesh=mesh,
        scratch_types=[
            pltpu.VMEM((_GROUP, _CHUNK), jnp.int32),
            pltpu.VMEM((_GROUP, _CHUNK), jnp.int32),
            pltpu.VMEM((_CHUNK, d), jnp.float32),
            pltpu.VMEM((_CHUNK, d), jnp.float32),
            pltpu.VMEM_SHARED((n_pad, d), jnp.float32),
            pltpu.SemaphoreType.DMA,
            pltpu.SemaphoreType.DMA,
        ],
    )
    def agg(x_hbm, src_hbm, dst_hbm, zero_hbm, out0, out1,
            src_i, dst_i, rows0, rows1, acc, sem0, sem1):
        core = lax.axis_index("c")
        sub = lax.axis_index("s")
        wid = core * _NS + sub
        rbase = sub * rows_pt

        pltpu.sync_copy(zero_hbm.at[pl.ds(rbase, rows_pt)],
                        acc.at[pl.ds(rbase, rows_pt)])
        plsc.subcore_barrier()

        def group_body(gr, carry):
            cbase = wid * cpw + gr * _GROUP
            pltpu.sync_copy(src_hbm.at[pl.ds(cbase, _GROUP)], src_i)
            pltpu.sync_copy(dst_hbm.at[pl.ds(cbase, _GROUP)], dst_i)

            pltpu.async_copy(x_hbm.at[src_i.at[0]], rows0, sem0)
            pltpu.async_copy(x_hbm.at[src_i.at[1]], rows1, sem1)

            def body(it, c2):
                g = it * 2
                pltpu.make_async_copy(
                    x_hbm.at[src_i.at[g]], rows0, sem0).wait()
                pltpu.sync_copy(rows0, acc.at[dst_i.at[g]], add=True)

                @pl.when(g + 2 < _GROUP)
                def _():
                    pltpu.async_copy(x_hbm.at[src_i.at[g + 2]], rows0, sem0)

                pltpu.make_async_copy(
                    x_hbm.at[src_i.at[g + 1]], rows1, sem1).wait()
                pltpu.sync_copy(rows1, acc.at[dst_i.at[g + 1]], add=True)

                @pl.when(g + 3 < _GROUP)
                def _():
                    pltpu.async_copy(x_hbm.at[src_i.at[g + 3]], rows1, sem1)

                return c2

            lax.fori_loop(0, _GROUP // 2, body, None)
            return carry

        lax.fori_loop(0, cpw // _GROUP, group_body, None)
        plsc.subcore_barrier()

        @pl.when(core == 0)
        def _():
            pltpu.sync_copy(acc.at[pl.ds(rbase, rows_pt)],
                            out0.at[pl.ds(rbase, rows_pt)])

        @pl.when(core == 1)
        def _():
            pltpu.sync_copy(acc.at[pl.ds(rbase, rows_pt)],
                            out1.at[pl.ds(rbase, rows_pt)])

    return agg(x, srcp, dstp, zeros)


def _tc_matmul(p0, p1, w, b, n):
    d_in, d_out = w.shape
    blk = 400
    grid = n // blk

    def mm(p0_ref, p1_ref, w_ref, b_ref, o_ref):
        a = p0_ref[...] + p1_ref[...]
        o_ref[...] = (
            jnp.dot(a, w_ref[...], preferred_element_type=jnp.float32)
            + b_ref[...]
        )

    return pl.pallas_call(
        mm,
        grid=(grid,),
        in_specs=[
            pl.BlockSpec((blk, d_in), lambda i: (i, 0)),
            pl.BlockSpec((blk, d_in), lambda i: (i, 0)),
            pl.BlockSpec((d_in, d_out), lambda i: (0, 0)),
            pl.BlockSpec((1, d_out), lambda i: (0, 0)),
        ],
        out_specs=pl.BlockSpec((blk, d_out), lambda i: (i, 0)),
        out_shape=jax.ShapeDtypeStruct((n, d_out), jnp.float32),
    )(p0, p1, w, b)


def kernel(x, edge_index, W, b):
    n, d = x.shape
    e = edge_index.shape[1]

    cpw = -(-e // (_NW * _CHUNK))
    cpw = -(-cpw // _GROUP) * _GROUP
    ep = cpw * _NW * _CHUNK
    rows_pt = -(-(n + 1) // _NS)
    rows_pt = -(-rows_pt // 8) * 8
    n_pad = rows_pt * _NS

    src = edge_index[0]
    dst = edge_index[1]
    pad = ep - e
    srcp = jnp.concatenate([src, jnp.zeros((pad,), jnp.int32)])
    dstp = jnp.concatenate([dst, jnp.full((pad,), n, jnp.int32)])
    srcp = srcp.reshape(ep // _CHUNK, _CHUNK)
    dstp = dstp.reshape(ep // _CHUNK, _CHUNK)
    zeros = jnp.zeros((n_pad, d), jnp.float32)

    p0, p1 = _sc_aggregate(n_pad, cpw, x, srcp, dstp, zeros)
    return _tc_matmul(p0, p1, W, b, n)
